# baseline (speedup 1.0000x reference)
import jax
import jax.numpy as jnp
from jax.experimental import pallas as pl
from jax.experimental.pallas import tpu as pltpu

_V = 256
_E = 128
_H = 128
_L = 2
_EPS = 1e-5
_BT = 512


def _layer_norm(x, gamma, beta):
    m1 = jnp.mean(x, axis=-1, keepdims=True)
    m2 = jnp.mean(x * x, axis=-1, keepdims=True)
    var = m2 - m1 * m1
    return (x - m1) * jax.lax.rsqrt(var + _EPS) * gamma + beta


def _cell(gates, c):
    one = jnp.bfloat16(1.0)
    t = jnp.tanh(gates.astype(jnp.bfloat16))
    t_i = (t[:, 0 * _H:1 * _H] + one).astype(jnp.float32)
    t_f = (t[:, 1 * _H:2 * _H] + one).astype(jnp.float32)
    g_g = t[:, 2 * _H:3 * _H].astype(jnp.float32)
    t_o = t[:, 3 * _H:4 * _H]
    c = 0.5 * (t_f * c + t_i * g_g)
    h = ((t_o + one) * jnp.tanh(c.astype(jnp.bfloat16))) * jnp.bfloat16(0.5)
    return h, c, t_o


def _h_f32(t_o, c):
    return (0.5 * (t_o.astype(jnp.float32) + 1.0)) * jnp.tanh(c)


def _lstm_kernel(idx_ref, table_ref, g_e_ref, b_e_ref,
                 w_ih0_ref, w_hh0_ref, b0_ref,
                 w_ih1_ref, w_hh1_ref, b1_ref,
                 g_h_ref, b_h_ref, w_fc_ref, b_fc_ref,
                 h0_ref, c0_ref,
                 logp_ref, h_n_ref, c_n_ref):
    bt, seq = idx_ref.shape

    col = jax.lax.broadcasted_iota(jnp.int32, (1, 4 * _H), 1)
    is_g = jnp.logical_and(col >= 2 * _H, col < 3 * _H)
    gsc = jnp.where(is_g, 1.0, 0.5).astype(jnp.float32)
    tln = _layer_norm(table_ref[...], g_e_ref[...], b_e_ref[...])
    g0 = (jnp.dot(tln.astype(jnp.bfloat16), w_ih0_ref[...].astype(jnp.bfloat16),
                  preferred_element_type=jnp.float32) + b0_ref[...]) * gsc
    w0 = jnp.concatenate([g0.astype(jnp.bfloat16),
                          (w_hh0_ref[...] * gsc).astype(jnp.bfloat16)], axis=0)
    w1 = jnp.concatenate([(w_ih1_ref[...] * gsc).astype(jnp.bfloat16),
                          (w_hh1_ref[...] * gsc).astype(jnp.bfloat16)], axis=0)
    b1 = b1_ref[...] * gsc

    idx = idx_ref[...]
    lane_iota = jax.lax.broadcasted_iota(jnp.int32, (bt, _V), 1)

    h0 = h0_ref[0].astype(jnp.bfloat16)
    h1 = h0_ref[1].astype(jnp.bfloat16)
    c0 = c0_ref[0]
    c1 = c0_ref[1]
    t_o0 = t_o1 = None

    for t in range(seq):
        onehot = (lane_iota == idx[:, t:t + 1]).astype(jnp.bfloat16)
        a0 = jnp.concatenate([onehot, h0], axis=1)
        gates0 = jnp.dot(a0, w0, preferred_element_type=jnp.float32)
        h0, c0, t_o0 = _cell(gates0, c0)
        a1 = jnp.concatenate([h0, h1], axis=1)
        gates1 = jnp.dot(a1, w1, preferred_element_type=jnp.float32) + b1
        h1, c1, t_o1 = _cell(gates1, c1)

    h0f = _h_f32(t_o0, c0)
    h1f = _h_f32(t_o1, c1)
    out = _layer_norm(h1f, g_h_ref[...], b_h_ref[...])
    logits = jnp.dot(out.astype(jnp.bfloat16), w_fc_ref[...].astype(jnp.bfloat16),
                     preferred_element_type=jnp.float32) + b_fc_ref[...]
    m = jnp.max(logits, axis=-1, keepdims=True)
    z = logits - m
    lse = jnp.log(jnp.sum(jnp.exp(z), axis=-1, keepdims=True))
    logp_ref[...] = z - lse

    h_n_ref[0] = h0f
    h_n_ref[1] = h1f
    c_n_ref[0] = c0
    c_n_ref[1] = c1


def kernel(input_seq, table, g_e, b_e, w_ih0, w_hh0, b0,
           w_ih1, w_hh1, b1, g_h, b_h, w_fc, b_fc, h0, c0):
    B, T = input_seq.shape
    bt = _BT if B % _BT == 0 else 8
    grid = (B // bt,)

    def full(shape):
        return pl.BlockSpec(shape, lambda b: (0,) * len(shape))

    in_specs = [
        pl.BlockSpec((bt, T), lambda b: (b, 0)),
        full((_V, _E)),
        full((1, _E)), full((1, _E)),
        full((_E, 4 * _H)), full((_H, 4 * _H)), full((1, 4 * _H)),
        full((_H, 4 * _H)), full((_H, 4 * _H)), full((1, 4 * _H)),
        full((1, _H)), full((1, _H)),
        full((_H, _V)), full((1, _V)),
        pl.BlockSpec((_L, bt, _H), lambda b: (0, b, 0)),
        pl.BlockSpec((_L, bt, _H), lambda b: (0, b, 0)),
    ]
    out_specs = (
        pl.BlockSpec((bt, _V), lambda b: (b, 0)),
        pl.BlockSpec((_L, bt, _H), lambda b: (0, b, 0)),
        pl.BlockSpec((_L, bt, _H), lambda b: (0, b, 0)),
    )
    out_shape = (
        jax.ShapeDtypeStruct((B, _V), jnp.float32),
        jax.ShapeDtypeStruct((_L, B, _H), jnp.float32),
        jax.ShapeDtypeStruct((_L, B, _H), jnp.float32),
    )

    logp, h_n, c_n = pl.pallas_call(
        _lstm_kernel,
        grid=grid,
        in_specs=in_specs,
        out_specs=out_specs,
        out_shape=out_shape,
        compiler_params=pltpu.CompilerParams(
            dimension_semantics=("parallel",)),
    )(input_seq.astype(jnp.int32), table, g_e, b_e,
      w_ih0, w_hh0, b0, w_ih1, w_hh1, b1,
      g_h, b_h, w_fc, b_fc, h0, c0)
    return logp, (h_n, c_n)


# speedup vs baseline: 1.1161x; 1.1161x over previous
import jax
import jax.numpy as jnp
from jax.experimental import pallas as pl
from jax.experimental.pallas import tpu as pltpu

_V = 256
_E = 128
_H = 128
_L = 2
_EPS = 1e-5
_BT = 512


def _layer_norm(x, gamma, beta):
    m1 = jnp.mean(x, axis=-1, keepdims=True)
    m2 = jnp.mean(x * x, axis=-1, keepdims=True)
    var = m2 - m1 * m1
    return (x - m1) * jax.lax.rsqrt(var + _EPS) * gamma + beta


def _cell(gates, c):
    t = jnp.tanh(gates)
    t_i = t[:, 0 * _H:1 * _H]
    t_f = t[:, 1 * _H:2 * _H]
    g_g = t[:, 2 * _H:3 * _H]
    t_o = t[:, 3 * _H:4 * _H]
    c = 0.5 * ((t_f + 1.0) * c + (t_i + 1.0) * g_g)
    h = (0.5 * (t_o + 1.0)) * jnp.tanh(c)
    return h, c


def _lstm_kernel(idx_ref, table_ref, g_e_ref, b_e_ref,
                 w_ih0_ref, w_hh0_ref, b0_ref,
                 w_ih1_ref, w_hh1_ref, b1_ref,
                 g_h_ref, b_h_ref, w_fc_ref, b_fc_ref,
                 h0_ref, c0_ref,
                 logp_ref, h_n_ref, c_n_ref):
    bt, seq = idx_ref.shape

    col = jax.lax.broadcasted_iota(jnp.int32, (1, 4 * _H), 1)
    is_g = jnp.logical_and(col >= 2 * _H, col < 3 * _H)
    gsc = jnp.where(is_g, 1.0, 0.5).astype(jnp.float32)
    tln = _layer_norm(table_ref[...], g_e_ref[...], b_e_ref[...])
    g0 = (jnp.dot(tln.astype(jnp.bfloat16), w_ih0_ref[...].astype(jnp.bfloat16),
                  preferred_element_type=jnp.float32) + b0_ref[...]) * gsc
    g0 = g0.astype(jnp.bfloat16)
    whh0 = (w_hh0_ref[...] * gsc).astype(jnp.bfloat16)
    w1 = jnp.concatenate([(w_ih1_ref[...] * gsc).astype(jnp.bfloat16),
                          (w_hh1_ref[...] * gsc).astype(jnp.bfloat16)], axis=0)
    b1 = b1_ref[...] * gsc

    idx = idx_ref[...]
    lane_iota = jax.lax.broadcasted_iota(jnp.int32, (bt, _V), 1)

    h0 = h0_ref[0]
    h1 = h0_ref[1]
    c0 = c0_ref[0]
    c1 = c0_ref[1]

    for t in range(seq):
        onehot = (lane_iota == idx[:, t:t + 1]).astype(jnp.bfloat16)
        emb0 = jnp.dot(onehot, g0, preferred_element_type=jnp.float32)
        gates0 = emb0 + jnp.dot(h0.astype(jnp.bfloat16), whh0,
                                preferred_element_type=jnp.float32)
        h0, c0 = _cell(gates0, c0)
        a1 = jnp.concatenate([h0.astype(jnp.bfloat16),
                              h1.astype(jnp.bfloat16)], axis=1)
        gates1 = jnp.dot(a1, w1, preferred_element_type=jnp.float32) + b1
        h1, c1 = _cell(gates1, c1)

    out = _layer_norm(h1, g_h_ref[...], b_h_ref[...])
    logits = jnp.dot(out.astype(jnp.bfloat16), w_fc_ref[...].astype(jnp.bfloat16),
                     preferred_element_type=jnp.float32) + b_fc_ref[...]
    m = jnp.max(logits, axis=-1, keepdims=True)
    z = logits - m
    lse = jnp.log(jnp.sum(jnp.exp(z), axis=-1, keepdims=True))
    logp_ref[...] = z - lse

    h_n_ref[0] = h0
    h_n_ref[1] = h1
    c_n_ref[0] = c0
    c_n_ref[1] = c1


def kernel(input_seq, table, g_e, b_e, w_ih0, w_hh0, b0,
           w_ih1, w_hh1, b1, g_h, b_h, w_fc, b_fc, h0, c0):
    B, T = input_seq.shape
    bt = _BT if B % _BT == 0 else 8
    grid = (B // bt,)

    def full(shape):
        return pl.BlockSpec(shape, lambda b: (0,) * len(shape))

    in_specs = [
        pl.BlockSpec((bt, T), lambda b: (b, 0)),
        full((_V, _E)),
        full((1, _E)), full((1, _E)),
        full((_E, 4 * _H)), full((_H, 4 * _H)), full((1, 4 * _H)),
        full((_H, 4 * _H)), full((_H, 4 * _H)), full((1, 4 * _H)),
        full((1, _H)), full((1, _H)),
        full((_H, _V)), full((1, _V)),
        pl.BlockSpec((_L, bt, _H), lambda b: (0, b, 0)),
        pl.BlockSpec((_L, bt, _H), lambda b: (0, b, 0)),
    ]
    out_specs = (
        pl.BlockSpec((bt, _V), lambda b: (b, 0)),
        pl.BlockSpec((_L, bt, _H), lambda b: (0, b, 0)),
        pl.BlockSpec((_L, bt, _H), lambda b: (0, b, 0)),
    )
    out_shape = (
        jax.ShapeDtypeStruct((B, _V), jnp.float32),
        jax.ShapeDtypeStruct((_L, B, _H), jnp.float32),
        jax.ShapeDtypeStruct((_L, B, _H), jnp.float32),
    )

    logp, h_n, c_n = pl.pallas_call(
        _lstm_kernel,
        grid=grid,
        in_specs=in_specs,
        out_specs=out_specs,
        out_shape=out_shape,
        compiler_params=pltpu.CompilerParams(
            dimension_semantics=("parallel",)),
    )(input_seq.astype(jnp.int32), table, g_e, b_e,
      w_ih0, w_hh0, b0, w_ih1, w_hh1, b1,
      g_h, b_h, w_fc, b_fc, h0, c0)
    return logp, (h_n, c_n)
